# baseline (speedup 1.0000x reference)
import jax
import jax.numpy as jnp
from jax import lax
from jax.experimental import pallas as pl
from jax.experimental.pallas import tpu as pltpu
from jax.experimental.pallas import tpu_sc as plsc

B = 16384
L = 20
VID = 100001
VIDP = 100008
VT = 10000
VG = 1001
VGP = 1008
NC, NS = 2, 16
NW = NC * NS
PB = B // NW

IDC = 3200
IDALIGN = 99968
IDTAIL = VID - IDALIGN
NMC = 384
NM_FULL = VT // NMC
NMALIGN = NM_FULL * NMC
NMTAIL = VT - NMALIGN

G = 4
CI = PB // G

_MESH = dict(core_axis_name="c", subcore_axis_name="s",
             num_cores=NC, num_subcores=NS)


def _detile_body(tokT, idT, nameT, idtail, nametail,
                 tokf, idf, namef,
                 tok_v, tokout_v, id_v, idout_v, nmin_v, nmout_v,
                 tail_v, ntail_v, sem_in, sem_out):
    wid = lax.axis_index("s") * NC + lax.axis_index("c")
    lane = lax.iota(jnp.int32, 16)

    pltpu.sync_copy(tokT.at[:, pl.ds(wid * PB, PB)], tok_v)

    def tok_body(c, carry):
        for j in range(L):
            tokout_v[pl.ds(j * PB + c * 16, 16)] = tok_v[j, pl.ds(c * 16, 16)]
        return carry

    lax.fori_loop(0, PB // 16, tok_body, 0)
    tok_cp = [pltpu.async_copy(tokout_v.at[pl.ds(j * PB, PB)],
                               tokf.at[pl.ds(j * B + wid * PB, PB)], sem_out)
              for j in range(L)]

    def _id_chunk(off, ncols):
        pltpu.sync_copy(idT.at[:, pl.ds(off, ncols)],
                        id_v.at[:, pl.ds(0, ncols)])
        d8 = lane & 7
        c2 = lane >> 3

        def id_body(c, carry):
            cc = c * 16
            for u in range(8):
                v = plsc.load_gather(
                    id_v, [d8, jnp.full((16,), cc + u * 2, jnp.int32) + c2])
                idout_v[pl.ds((cc + u * 2) * 8, 16)] = v
            return carry

        lax.fori_loop(0, ncols // 16, id_body, 0)
        pltpu.sync_copy(idout_v.at[pl.ds(0, ncols * 8)],
                        idf.at[pl.ds(off * 8, ncols * 8)])

    @pl.when(wid < 31)
    def _id_full():
        _id_chunk(wid * IDC, IDC)

    @pl.when(wid == 31)
    def _id_last():
        _id_chunk(31 * IDC, IDALIGN - 31 * IDC)
        pltpu.sync_copy(idtail, tail_v)
        pltpu.sync_copy(tail_v, idf.at[pl.ds(IDALIGN * 8, IDTAIL * 8)])

    @pl.when(wid < NM_FULL)
    def _name_full():
        off_cols = wid * NMC
        pltpu.sync_copy(nameT.at[:, pl.ds(off_cols, NMC)], nmin_v)

        def row_body(t, carry):
            t0 = t * 8
            for u in range(8):
                v = plsc.load_gather(
                    nmin_v, [lane, jnp.full((16,), u, jnp.int32) + t0])
                nmout_v[pl.ds((t0 + u) * 16, 16)] = v
            return carry

        lax.fori_loop(0, NMC // 8, row_body, 0)
        pltpu.sync_copy(nmout_v, namef.at[pl.ds(off_cols * 16, NMC * 16)])

    @pl.when(wid == NM_FULL)
    def _name_last():
        pltpu.sync_copy(nametail, ntail_v)
        pltpu.sync_copy(ntail_v, namef.at[pl.ds(NMALIGN * 16, NMTAIL * 16)])

    for cp in tok_cp:
        cp.wait()


@jax.jit
def _detile(tokT, idT, nameT, idtail, nametail):
    mesh = plsc.VectorSubcoreMesh(**_MESH)
    return pl.kernel(
        _detile_body,
        out_type=(
            jax.ShapeDtypeStruct((B * L,), jnp.int32),
            jax.ShapeDtypeStruct((VID * 8,), jnp.float32),
            jax.ShapeDtypeStruct((VT * 16,), jnp.float32),
        ),
        mesh=mesh,
        scratch_types=[
            pltpu.VMEM((L, PB), jnp.int32),
            pltpu.VMEM((L * PB,), jnp.int32),
            pltpu.VMEM((8, IDC), jnp.float32),
            pltpu.VMEM((8 * IDC,), jnp.float32),
            pltpu.VMEM((16, NMC), jnp.float32),
            pltpu.VMEM((NMC * 16,), jnp.float32),
            pltpu.VMEM((IDTAIL * 8,), jnp.float32),
            pltpu.VMEM((NMTAIL * 16,), jnp.float32),
            pltpu.SemaphoreType.DMA,
            pltpu.SemaphoreType.DMA,
        ],
        compiler_params=pltpu.CompilerParams(
            needs_layout_passes=False, use_tc_tiling_on_sc=True),
    )(tokT, idT, nameT, idtail, nametail)


def _gather_body(idsf, tok2d, gicsids, id2d, name2d, gicsf, out,
                 tok_v, idid_v, gid_v, gicstab_v, idrow_v,
                 rows0, rows1, out_v, row0_v, rec_v, nz_v,
                 sem_id, sem_n0, sem_n1, sem_out):
    wid = lax.axis_index("s") * NC + lax.axis_index("c")
    base = wid * PB
    rows = (rows0, rows1)
    sem_n = (sem_n0, sem_n1)
    lane = lax.iota(jnp.int32, 16)
    lane_lo = lane < 8
    col8 = lane & 7

    pltpu.sync_copy(tok2d.at[:, pl.ds(base, PB)], tok_v)
    pltpu.sync_copy(idsf.at[pl.ds(base, PB)], idid_v)
    pltpu.sync_copy(gicsids.at[pl.ds(base, PB)], gid_v)
    pltpu.sync_copy(gicsf, gicstab_v)
    pltpu.sync_copy(name2d.at[pl.ds(0, 1)], row0_v)

    id_cp = [pltpu.async_copy(
        id2d.at[idid_v.at[pl.ds(t * 128, 128)]],
        idrow_v.at[pl.ds(t * 128, 128)], sem_id)
        for t in range(PB // 128)]

    def cnt_body(k, carry):
        i0 = k * 16
        cnt = jnp.zeros((16,), jnp.float32)
        for j in range(L):
            cnt = cnt + jnp.where(tok_v[j, pl.ds(i0, 16)] != 0, 1.0, 0.0)
        rec_v[pl.ds(i0, 16)] = 1.0 / jnp.maximum(cnt, jnp.float32(1.0))
        nz_v[pl.ds(i0, 16)] = jnp.float32(L) - cnt
        return carry

    lax.fori_loop(0, PB // 16, cnt_body, 0)

    def fire_group(g):
        buf = g % 2
        return [pltpu.async_copy(
            name2d.at[tok_v.at[j, pl.ds(g * CI, CI)]],
            rows[buf].at[pl.ds(j * CI, CI)], sem_n[buf])
            for j in range(L)]

    pend = fire_group(0)
    row0 = row0_v[0, :]
    for cp in id_cp:
        cp.wait()
    mrow = jnp.where(lane_lo, lane, lane + 16)
    nrow = lane + 8

    for g in range(G):
        buf = g % 2
        nxt = fire_group(g + 1) if g + 1 < G else None
        for cp in pend:
            cp.wait()
        pend = nxt

        def item_body(k, carry, g=g, buf=buf):
            for u in range(2):
                li = k * 2 + u
                gi = g * CI + li
                r = [rows[buf][j * CI + li, :] for j in range(L)]
                while len(r) > 1:
                    r = [a + b for a, b in zip(r[::2], r[1::2])] + \
                        ([r[-1]] if len(r) % 2 else [])
                acc = r[0]
                gsp = jnp.full((16,), gi, dtype=jnp.int32)
                nz = plsc.load_gather(nz_v, [gsp])
                rec = plsc.load_gather(rec_v, [gsp])
                name = (acc - nz * row0) * rec
                idv16 = plsc.load_gather(idrow_v, [gsp, col8])
                gsplat = plsc.load_gather(gid_v, [gsp])
                gicsv = plsc.load_gather(gicstab_v, [col8 * VG + gsplat])
                merged = jnp.where(lane_lo, idv16, gicsv)
                plsc.store_scatter(out_v, [mrow, gsp], merged)
                plsc.store_scatter(out_v, [nrow, gsp], name)
            return carry

        lax.fori_loop(0, CI // 2, item_body, 0)

    pltpu.sync_copy(out_v, out.at[:, pl.ds(base, PB)])


@jax.jit
def _gather(idsf, tok2d, gicsids, id2d, name2d, gicsf):
    mesh = plsc.VectorSubcoreMesh(**_MESH)
    return pl.kernel(
        _gather_body,
        out_type=jax.ShapeDtypeStruct((32, B), jnp.float32),
        mesh=mesh,
        scratch_types=[
            pltpu.VMEM((L, PB), jnp.int32),
            pltpu.VMEM((PB,), jnp.int32),
            pltpu.VMEM((PB,), jnp.int32),
            pltpu.VMEM((8 * VG,), jnp.float32),
            pltpu.VMEM((PB, 8), jnp.float32),
            pltpu.VMEM((CI * L, 16), jnp.float32),
            pltpu.VMEM((CI * L, 16), jnp.float32),
            pltpu.VMEM((32, PB), jnp.float32),
            pltpu.VMEM((1, 16), jnp.float32),
            pltpu.VMEM((PB,), jnp.float32),
            pltpu.VMEM((PB,), jnp.float32),
            pltpu.SemaphoreType.DMA,
            pltpu.SemaphoreType.DMA,
            pltpu.SemaphoreType.DMA,
            pltpu.SemaphoreType.DMA,
        ],
        compiler_params=pltpu.CompilerParams(
            needs_layout_passes=False, use_tc_tiling_on_sc=False),
    )(idsf, tok2d, gicsids, id2d, name2d, gicsf)


def kernel(item_id, item_name_tokens, item_gics, emb_id, emb_name, emb_gics):
    idsf = item_id.astype(jnp.int32)
    gicsids = item_gics.astype(jnp.int32)
    idtail = emb_id[IDALIGN:].reshape(-1)
    nametail = emb_name[NMALIGN:].reshape(-1)
    gicsf = emb_gics.T.reshape(-1)
    tokf, idf, namef = _detile(
        item_name_tokens.astype(jnp.int32).T, emb_id.T, emb_name.T,
        idtail, nametail)
    outT = _gather(idsf, tokf.reshape(L, B), gicsids, idf.reshape(VID, 8),
                   namef.reshape(VT, 16), gicsf)
    return outT.T


# speedup vs baseline: 1.0088x; 1.0088x over previous
import jax
import jax.numpy as jnp
from jax import lax
from jax.experimental import pallas as pl
from jax.experimental.pallas import tpu as pltpu
from jax.experimental.pallas import tpu_sc as plsc

B = 16384
L = 20
VID = 100001
VIDP = 100008
VT = 10000
VG = 1001
VGP = 1008
NC, NS = 2, 16
NW = NC * NS
PB = B // NW

IDC = 3200
IDALIGN = 99968
IDTAIL = VID - IDALIGN
NMC = 384
NM_FULL = VT // NMC
NMALIGN = NM_FULL * NMC
NMTAIL = VT - NMALIGN

G = 8
CI = PB // G

_MESH = dict(core_axis_name="c", subcore_axis_name="s",
             num_cores=NC, num_subcores=NS)


def _detile_body(tokT, idT, nameT, idtail, nametail,
                 tokf, idf, namef,
                 tok_v, tokout_v, id_v, idout_v, nmin_v, nmout_v,
                 tail_v, ntail_v, sem_in, sem_out):
    wid = lax.axis_index("s") * NC + lax.axis_index("c")
    lane = lax.iota(jnp.int32, 16)

    pltpu.sync_copy(tokT.at[:, pl.ds(wid * PB, PB)], tok_v)

    def tok_body(c, carry):
        for j in range(L):
            tokout_v[pl.ds(j * PB + c * 16, 16)] = tok_v[j, pl.ds(c * 16, 16)]
        return carry

    lax.fori_loop(0, PB // 16, tok_body, 0)
    tok_cp = [pltpu.async_copy(tokout_v.at[pl.ds(j * PB, PB)],
                               tokf.at[pl.ds(j * B + wid * PB, PB)], sem_out)
              for j in range(L)]

    def _id_chunk(off, ncols):
        pltpu.sync_copy(idT.at[:, pl.ds(off, ncols)],
                        id_v.at[:, pl.ds(0, ncols)])
        d8 = lane & 7
        c2 = lane >> 3

        def id_body(c, carry):
            cc = c * 16
            for u in range(8):
                v = plsc.load_gather(
                    id_v, [d8, jnp.full((16,), cc + u * 2, jnp.int32) + c2])
                idout_v[pl.ds((cc + u * 2) * 8, 16)] = v
            return carry

        lax.fori_loop(0, ncols // 16, id_body, 0)
        pltpu.sync_copy(idout_v.at[pl.ds(0, ncols * 8)],
                        idf.at[pl.ds(off * 8, ncols * 8)])

    @pl.when(wid < 31)
    def _id_full():
        _id_chunk(wid * IDC, IDC)

    @pl.when(wid == 31)
    def _id_last():
        _id_chunk(31 * IDC, IDALIGN - 31 * IDC)
        pltpu.sync_copy(idtail, tail_v)
        pltpu.sync_copy(tail_v, idf.at[pl.ds(IDALIGN * 8, IDTAIL * 8)])

    @pl.when(wid < NM_FULL)
    def _name_full():
        off_cols = wid * NMC
        pltpu.sync_copy(nameT.at[:, pl.ds(off_cols, NMC)], nmin_v)

        def row_body(t, carry):
            t0 = t * 8
            for u in range(8):
                v = plsc.load_gather(
                    nmin_v, [lane, jnp.full((16,), u, jnp.int32) + t0])
                nmout_v[pl.ds((t0 + u) * 16, 16)] = v
            return carry

        lax.fori_loop(0, NMC // 8, row_body, 0)
        pltpu.sync_copy(nmout_v, namef.at[pl.ds(off_cols * 16, NMC * 16)])

    @pl.when(wid == NM_FULL)
    def _name_last():
        pltpu.sync_copy(nametail, ntail_v)
        pltpu.sync_copy(ntail_v, namef.at[pl.ds(NMALIGN * 16, NMTAIL * 16)])

    for cp in tok_cp:
        cp.wait()


@jax.jit
def _detile(tokT, idT, nameT, idtail, nametail):
    mesh = plsc.VectorSubcoreMesh(**_MESH)
    return pl.kernel(
        _detile_body,
        out_type=(
            jax.ShapeDtypeStruct((B * L,), jnp.int32),
            jax.ShapeDtypeStruct((VID * 8,), jnp.float32),
            jax.ShapeDtypeStruct((VT * 16,), jnp.float32),
        ),
        mesh=mesh,
        scratch_types=[
            pltpu.VMEM((L, PB), jnp.int32),
            pltpu.VMEM((L * PB,), jnp.int32),
            pltpu.VMEM((8, IDC), jnp.float32),
            pltpu.VMEM((8 * IDC,), jnp.float32),
            pltpu.VMEM((16, NMC), jnp.float32),
            pltpu.VMEM((NMC * 16,), jnp.float32),
            pltpu.VMEM((IDTAIL * 8,), jnp.float32),
            pltpu.VMEM((NMTAIL * 16,), jnp.float32),
            pltpu.SemaphoreType.DMA,
            pltpu.SemaphoreType.DMA,
        ],
        compiler_params=pltpu.CompilerParams(
            needs_layout_passes=False, use_tc_tiling_on_sc=True),
    )(tokT, idT, nameT, idtail, nametail)


def _gather_body(idsf, tok2d, gicsids, id2d, name2d, gicsf, out,
                 tok_v, idid_v, gid_v, gicstab_v, idrow_v,
                 rows0, rows1, out_v, row0_v, rec_v, nz_v,
                 sem_id, sem_n0, sem_n1, sem_out, nmtab):
    wid = lax.axis_index("s") * NC + lax.axis_index("c")
    sid = lax.axis_index("s")
    base = wid * PB
    rows = (rows0, rows1)
    sem_n = (sem_n0, sem_n1)
    lane = lax.iota(jnp.int32, 16)
    lane_lo = lane < 8
    col8 = lane & 7

    nm_rows = VT // NS
    pltpu.sync_copy(name2d.at[pl.ds(sid * nm_rows, nm_rows)],
                    nmtab.at[pl.ds(sid * nm_rows, nm_rows)])

    pltpu.sync_copy(tok2d.at[:, pl.ds(base, PB)], tok_v)
    pltpu.sync_copy(idsf.at[pl.ds(base, PB)], idid_v)
    pltpu.sync_copy(gicsids.at[pl.ds(base, PB)], gid_v)
    pltpu.sync_copy(gicsf, gicstab_v)
    pltpu.sync_copy(name2d.at[pl.ds(0, 1)], row0_v)
    plsc.subcore_barrier()

    id_cp = [pltpu.async_copy(
        id2d.at[idid_v.at[pl.ds(t * 128, 128)]],
        idrow_v.at[pl.ds(t * 128, 128)], sem_id)
        for t in range(PB // 128)]

    def cnt_body(k, carry):
        i0 = k * 16
        cnt = jnp.zeros((16,), jnp.float32)
        for j in range(L):
            cnt = cnt + jnp.where(tok_v[j, pl.ds(i0, 16)] != 0, 1.0, 0.0)
        rec_v[pl.ds(i0, 16)] = 1.0 / jnp.maximum(cnt, jnp.float32(1.0))
        nz_v[pl.ds(i0, 16)] = jnp.float32(L) - cnt
        return carry

    lax.fori_loop(0, PB // 16, cnt_body, 0)

    def fire_group(g):
        buf = g % 2
        return [pltpu.async_copy(
            nmtab.at[tok_v.at[j, pl.ds(g * CI, CI)]],
            rows[buf].at[pl.ds(j * CI, CI)], sem_n[buf])
            for j in range(L)]

    pend = fire_group(0)
    row0 = row0_v[0, :]
    for cp in id_cp:
        cp.wait()
    mrow = jnp.where(lane_lo, lane, lane + 16)
    nrow = lane + 8

    for g in range(G):
        buf = g % 2
        nxt = fire_group(g + 1) if g + 1 < G else None
        for cp in pend:
            cp.wait()
        pend = nxt

        def item_body(k, carry, g=g, buf=buf):
            for u in range(2):
                li = k * 2 + u
                gi = g * CI + li
                r = [rows[buf][j * CI + li, :] for j in range(L)]
                while len(r) > 1:
                    r = [a + b for a, b in zip(r[::2], r[1::2])] + \
                        ([r[-1]] if len(r) % 2 else [])
                acc = r[0]
                gsp = jnp.full((16,), gi, dtype=jnp.int32)
                nz = plsc.load_gather(nz_v, [gsp])
                rec = plsc.load_gather(rec_v, [gsp])
                name = (acc - nz * row0) * rec
                idv16 = plsc.load_gather(idrow_v, [gsp, col8])
                gsplat = plsc.load_gather(gid_v, [gsp])
                gicsv = plsc.load_gather(gicstab_v, [col8 * VG + gsplat])
                merged = jnp.where(lane_lo, idv16, gicsv)
                plsc.store_scatter(out_v, [mrow, gsp], merged)
                plsc.store_scatter(out_v, [nrow, gsp], name)
            return carry

        lax.fori_loop(0, CI // 2, item_body, 0)

    pltpu.sync_copy(out_v, out.at[:, pl.ds(base, PB)])


@jax.jit
def _gather(idsf, tok2d, gicsids, id2d, name2d, gicsf):
    mesh = plsc.VectorSubcoreMesh(**_MESH)
    return pl.kernel(
        _gather_body,
        out_type=jax.ShapeDtypeStruct((32, B), jnp.float32),
        mesh=mesh,
        scratch_types=[
            pltpu.VMEM((L, PB), jnp.int32),
            pltpu.VMEM((PB,), jnp.int32),
            pltpu.VMEM((PB,), jnp.int32),
            pltpu.VMEM((8 * VG,), jnp.float32),
            pltpu.VMEM((PB, 8), jnp.float32),
            pltpu.VMEM((CI * L, 16), jnp.float32),
            pltpu.VMEM((CI * L, 16), jnp.float32),
            pltpu.VMEM((32, PB), jnp.float32),
            pltpu.VMEM((1, 16), jnp.float32),
            pltpu.VMEM((PB,), jnp.float32),
            pltpu.VMEM((PB,), jnp.float32),
            pltpu.SemaphoreType.DMA,
            pltpu.SemaphoreType.DMA,
            pltpu.SemaphoreType.DMA,
            pltpu.SemaphoreType.DMA,
            pltpu.MemorySpace.VMEM_SHARED((VT, 16), jnp.float32),
        ],
        compiler_params=pltpu.CompilerParams(
            needs_layout_passes=False, use_tc_tiling_on_sc=False),
    )(idsf, tok2d, gicsids, id2d, name2d, gicsf)


def kernel(item_id, item_name_tokens, item_gics, emb_id, emb_name, emb_gics):
    idsf = item_id.astype(jnp.int32)
    gicsids = item_gics.astype(jnp.int32)
    idtail = emb_id[IDALIGN:].reshape(-1)
    nametail = emb_name[NMALIGN:].reshape(-1)
    gicsf = emb_gics.T.reshape(-1)
    tokf, idf, namef = _detile(
        item_name_tokens.astype(jnp.int32).T, emb_id.T, emb_name.T,
        idtail, nametail)
    outT = _gather(idsf, tokf.reshape(L, B), gicsids, idf.reshape(VID, 8),
                   namef.reshape(VT, 16), gicsf)
    return outT.T


# speedup vs baseline: 1.0844x; 1.0750x over previous
import jax
import jax.numpy as jnp
from jax import lax
from jax.experimental import pallas as pl
from jax.experimental.pallas import tpu as pltpu
from jax.experimental.pallas import tpu_sc as plsc

B = 16384
L = 20
VID = 100001
VIDP = 100008
VT = 10000
VG = 1001
VGP = 1008
NC, NS = 2, 16
NW = NC * NS
PB = B // NW

IDC = 3200
IDALIGN = 99968
IDTAIL = VID - IDALIGN
NMC = 384
NM_FULL = VT // NMC
NMALIGN = NM_FULL * NMC
NMTAIL = VT - NMALIGN

G = 8
CI = PB // G

_MESH = dict(core_axis_name="c", subcore_axis_name="s",
             num_cores=NC, num_subcores=NS)


def _detile_body(tokT, idT, nameT, idtail, nametail,
                 tokf, idf, namef,
                 tok_v, tokout_v, id_v, idout_v, nmin_v, nmout_v,
                 tail_v, ntail_v, sem_in, sem_out):
    wid = lax.axis_index("s") * NC + lax.axis_index("c")
    lane = lax.iota(jnp.int32, 16)

    pltpu.sync_copy(tokT.at[:, pl.ds(wid * PB, PB)], tok_v)

    def tok_body(c, carry):
        for j in range(L):
            tokout_v[pl.ds(j * PB + c * 16, 16)] = tok_v[j, pl.ds(c * 16, 16)]
        return carry

    lax.fori_loop(0, PB // 16, tok_body, 0)
    tok_cp = [pltpu.async_copy(tokout_v.at[pl.ds(j * PB, PB)],
                               tokf.at[pl.ds(j * B + wid * PB, PB)], sem_out)
              for j in range(L)]

    def _id_chunk(off, ncols):
        pltpu.sync_copy(idT.at[:, pl.ds(off, ncols)],
                        id_v.at[:, pl.ds(0, ncols)])
        d8 = lane & 7
        c2 = lane >> 3

        def id_body(c, carry):
            cc = c * 16
            for u in range(8):
                v = plsc.load_gather(
                    id_v, [d8, jnp.full((16,), cc + u * 2, jnp.int32) + c2])
                idout_v[pl.ds((cc + u * 2) * 8, 16)] = v
            return carry

        lax.fori_loop(0, ncols // 16, id_body, 0)
        pltpu.sync_copy(idout_v.at[pl.ds(0, ncols * 8)],
                        idf.at[pl.ds(off * 8, ncols * 8)])

    @pl.when(wid < 31)
    def _id_full():
        _id_chunk(wid * IDC, IDC)

    @pl.when(wid == 31)
    def _id_last():
        _id_chunk(31 * IDC, IDALIGN - 31 * IDC)
        pltpu.sync_copy(idtail, tail_v)
        pltpu.sync_copy(tail_v, idf.at[pl.ds(IDALIGN * 8, IDTAIL * 8)])

    @pl.when(wid < NM_FULL)
    def _name_full():
        off_cols = wid * NMC
        pltpu.sync_copy(nameT.at[:, pl.ds(off_cols, NMC)], nmin_v)

        def row_body(t, carry):
            t0 = t * 8
            for u in range(8):
                v = plsc.load_gather(
                    nmin_v, [lane, jnp.full((16,), u, jnp.int32) + t0])
                nmout_v[pl.ds((t0 + u) * 16, 16)] = v
            return carry

        lax.fori_loop(0, NMC // 8, row_body, 0)
        pltpu.sync_copy(nmout_v, namef.at[pl.ds(off_cols * 16, NMC * 16)])

    @pl.when(wid == NM_FULL)
    def _name_last():
        pltpu.sync_copy(nametail, ntail_v)
        pltpu.sync_copy(ntail_v, namef.at[pl.ds(NMALIGN * 16, NMTAIL * 16)])

    for cp in tok_cp:
        cp.wait()


@jax.jit
def _detile(tokT, idT, nameT, idtail, nametail):
    mesh = plsc.VectorSubcoreMesh(**_MESH)
    return pl.kernel(
        _detile_body,
        out_type=(
            jax.ShapeDtypeStruct((B * L,), jnp.int32),
            jax.ShapeDtypeStruct((VID * 8,), jnp.float32),
            jax.ShapeDtypeStruct((VT * 16,), jnp.float32),
        ),
        mesh=mesh,
        scratch_types=[
            pltpu.VMEM((L, PB), jnp.int32),
            pltpu.VMEM((L * PB,), jnp.int32),
            pltpu.VMEM((8, IDC), jnp.float32),
            pltpu.VMEM((8 * IDC,), jnp.float32),
            pltpu.VMEM((16, NMC), jnp.float32),
            pltpu.VMEM((NMC * 16,), jnp.float32),
            pltpu.VMEM((IDTAIL * 8,), jnp.float32),
            pltpu.VMEM((NMTAIL * 16,), jnp.float32),
            pltpu.SemaphoreType.DMA,
            pltpu.SemaphoreType.DMA,
        ],
        compiler_params=pltpu.CompilerParams(
            needs_layout_passes=False, use_tc_tiling_on_sc=True),
    )(tokT, idT, nameT, idtail, nametail)


def _gather_body(idsf, tok2d, gicsids, id2d, name2d, gicsf, out,
                 tok_v, idid_v, gid_v, gicstab_v, idrow_v,
                 rows0, rows1, out_v, row0_v, rec_v, nz_v,
                 sem_id, sem_n0, sem_n1, sem_out, nmtab):
    wid = lax.axis_index("s") * NC + lax.axis_index("c")
    sid = lax.axis_index("s")
    base = wid * PB
    rows = (rows0, rows1)
    sem_n = (sem_n0, sem_n1)
    lane = lax.iota(jnp.int32, 16)
    lane_lo = lane < 8
    col8 = lane & 7

    nm_rows = VT // NS
    pltpu.sync_copy(name2d.at[pl.ds(sid * nm_rows, nm_rows)],
                    nmtab.at[pl.ds(sid * nm_rows, nm_rows)])

    pltpu.sync_copy(tok2d.at[:, pl.ds(base, PB)], tok_v)
    pltpu.sync_copy(idsf.at[pl.ds(base, PB)], idid_v)
    pltpu.sync_copy(gicsids.at[pl.ds(base, PB)], gid_v)
    pltpu.sync_copy(gicsf, gicstab_v)
    pltpu.sync_copy(name2d.at[pl.ds(0, 1)], row0_v)
    plsc.subcore_barrier()

    id_cp = [pltpu.async_copy(
        id2d.at[idid_v.at[pl.ds(t * 128, 128)]],
        idrow_v.at[pl.ds(t * 128, 128)], sem_id)
        for t in range(PB // 128)]

    def cnt_body(k, carry):
        i0 = k * 16
        cnt = jnp.zeros((16,), jnp.float32)
        for j in range(L):
            cnt = cnt + jnp.where(tok_v[j, pl.ds(i0, 16)] != 0, 1.0, 0.0)
        rec_v[pl.ds(i0, 16)] = 1.0 / jnp.maximum(cnt, jnp.float32(1.0))
        nz_v[pl.ds(i0, 16)] = jnp.float32(L) - cnt
        return carry

    lax.fori_loop(0, PB // 16, cnt_body, 0)

    row0 = row0_v[0, :]
    for cp in id_cp:
        cp.wait()
    mrow = jnp.where(lane_lo, lane, lane + 16)
    nrow = lane + 8

    def process_group(g, buf):
        def item_body(k, carry, g=g, buf=buf):
            for u in range(2):
                li = k * 2 + u
                gi = g * CI + li
                acc = rows[buf][li, :]
                gsp = jnp.full((16,), gi, dtype=jnp.int32)
                nz = plsc.load_gather(nz_v, [gsp])
                rec = plsc.load_gather(rec_v, [gsp])
                name = (acc - nz * row0) * rec
                idv16 = plsc.load_gather(idrow_v, [gsp, col8])
                gsplat = plsc.load_gather(gid_v, [gsp])
                gicsv = plsc.load_gather(gicstab_v, [col8 * VG + gsplat])
                merged = jnp.where(lane_lo, idv16, gicsv)
                plsc.store_scatter(out_v, [mrow, gsp], merged)
                plsc.store_scatter(out_v, [nrow, gsp], name)
            return carry

        lax.fori_loop(0, CI // 2, item_body, 0)

    pend = None
    for g in range(G):
        buf = g % 2
        cp0 = pltpu.async_copy(
            nmtab.at[tok_v.at[0, pl.ds(g * CI, CI)]],
            rows[buf], sem_n[buf])
        cp0.wait()
        adds = [pltpu.async_copy(
            nmtab.at[tok_v.at[j, pl.ds(g * CI, CI)]],
            rows[buf], sem_n[buf], add=True)
            for j in range(1, L)]
        if g > 0:
            process_group(g - 1, 1 - buf)
        for cp in adds:
            cp.wait()
        pend = buf
    process_group(G - 1, pend)

    pltpu.sync_copy(out_v, out.at[:, pl.ds(base, PB)])


@jax.jit
def _gather(idsf, tok2d, gicsids, id2d, name2d, gicsf):
    mesh = plsc.VectorSubcoreMesh(**_MESH)
    return pl.kernel(
        _gather_body,
        out_type=jax.ShapeDtypeStruct((32, B), jnp.float32),
        mesh=mesh,
        scratch_types=[
            pltpu.VMEM((L, PB), jnp.int32),
            pltpu.VMEM((PB,), jnp.int32),
            pltpu.VMEM((PB,), jnp.int32),
            pltpu.VMEM((8 * VG,), jnp.float32),
            pltpu.VMEM((PB, 8), jnp.float32),
            pltpu.VMEM((CI, 16), jnp.float32),
            pltpu.VMEM((CI, 16), jnp.float32),
            pltpu.VMEM((32, PB), jnp.float32),
            pltpu.VMEM((1, 16), jnp.float32),
            pltpu.VMEM((PB,), jnp.float32),
            pltpu.VMEM((PB,), jnp.float32),
            pltpu.SemaphoreType.DMA,
            pltpu.SemaphoreType.DMA,
            pltpu.SemaphoreType.DMA,
            pltpu.SemaphoreType.DMA,
            pltpu.MemorySpace.VMEM_SHARED((VT, 16), jnp.float32),
        ],
        compiler_params=pltpu.CompilerParams(
            needs_layout_passes=False, use_tc_tiling_on_sc=False),
    )(idsf, tok2d, gicsids, id2d, name2d, gicsf)


def kernel(item_id, item_name_tokens, item_gics, emb_id, emb_name, emb_gics):
    idsf = item_id.astype(jnp.int32)
    gicsids = item_gics.astype(jnp.int32)
    idtail = emb_id[IDALIGN:].reshape(-1)
    nametail = emb_name[NMALIGN:].reshape(-1)
    gicsf = emb_gics.T.reshape(-1)
    tokf, idf, namef = _detile(
        item_name_tokens.astype(jnp.int32).T, emb_id.T, emb_name.T,
        idtail, nametail)
    outT = _gather(idsf, tokf.reshape(L, B), gicsids, idf.reshape(VID, 8),
                   namef.reshape(VT, 16), gicsf)
    return outT.T
